# baseline (speedup 1.0000x reference)
import functools

import jax
import jax.numpy as jnp
from jax import lax
from jax.experimental import pallas as pl
from jax.experimental.pallas import tpu as pltpu
from jax.experimental.pallas import tpu_sc as plsc

NUM_FREQ = 3
ROWS_PER_BLOCK = 2048
_NC, _NS = 2, 16


def _sc_lookup_body(idx_hbm, emb_hbm, out_hbm, idx_v, row_v, sem):
    wid = lax.axis_index("s") * _NC + lax.axis_index("c")

    @pl.when(wid == 0)
    def _():
        pltpu.sync_copy(idx_hbm, idx_v)
        pltpu.async_copy(emb_hbm.at[idx_v], row_v, sem).wait()
        pltpu.sync_copy(row_v, out_hbm)


def _sc_lookup(idx_arr, freq_embeddings):
    d = freq_embeddings.shape[-1]
    mesh = plsc.VectorSubcoreMesh(core_axis_name="c", subcore_axis_name="s")
    f = pl.kernel(
        _sc_lookup_body,
        out_type=jax.ShapeDtypeStruct((1, d), jnp.float32),
        mesh=mesh,
        scratch_types=[
            pltpu.VMEM((1,), jnp.int32),
            pltpu.VMEM((1, d), jnp.float32),
            pltpu.SemaphoreType.DMA,
        ],
    )
    return f(idx_arr, freq_embeddings)


def _tc_add_body(x_ref, row_ref, o_ref):
    o_ref[...] = x_ref[...] + row_ref[...]


def kernel(x, freq_idx, freq_embeddings):
    orig_shape = x.shape
    d = orig_shape[-1]
    x2 = x.reshape(-1, d)
    n_rows = x2.shape[0]
    rpb = ROWS_PER_BLOCK
    idx_arr = jnp.asarray(freq_idx, jnp.int32).reshape(1)

    row = _sc_lookup(idx_arr, freq_embeddings)

    out = pl.pallas_call(
        _tc_add_body,
        grid=(n_rows // rpb,),
        in_specs=[
            pl.BlockSpec((rpb, d), lambda i: (i, 0)),
            pl.BlockSpec((1, d), lambda i: (0, 0)),
        ],
        out_specs=pl.BlockSpec((rpb, d), lambda i: (i, 0)),
        out_shape=jax.ShapeDtypeStruct((n_rows, d), x.dtype),
        compiler_params=pltpu.CompilerParams(
            dimension_semantics=("arbitrary",),
        ),
    )(x2, row)
    return out.reshape(orig_shape)


# speedup vs baseline: 1.0075x; 1.0075x over previous
import functools

import jax
import jax.numpy as jnp
from jax import lax
from jax.experimental import pallas as pl
from jax.experimental.pallas import tpu as pltpu
from jax.experimental.pallas import tpu_sc as plsc

NUM_FREQ = 3
ROWS_PER_BLOCK = 2048
_NC, _NS = 2, 16


def _sc_lookup_body(idx_hbm, emb_hbm, out_hbm, idx_v, row_v, sem):
    pltpu.sync_copy(idx_hbm, idx_v)
    pltpu.async_copy(emb_hbm.at[idx_v], row_v, sem).wait()
    pltpu.sync_copy(row_v, out_hbm)


def _sc_lookup(idx_arr, freq_embeddings):
    d = freq_embeddings.shape[-1]
    mesh = plsc.VectorSubcoreMesh(
        core_axis_name="c", subcore_axis_name="s", num_cores=1, num_subcores=1
    )
    f = pl.kernel(
        _sc_lookup_body,
        out_type=jax.ShapeDtypeStruct((1, d), jnp.float32),
        mesh=mesh,
        scratch_types=[
            pltpu.VMEM((1,), jnp.int32),
            pltpu.VMEM((1, d), jnp.float32),
            pltpu.SemaphoreType.DMA,
        ],
    )
    return f(idx_arr, freq_embeddings)


def _tc_add_body(x_ref, row_ref, o_ref):
    o_ref[...] = x_ref[...] + row_ref[...]


def kernel(x, freq_idx, freq_embeddings):
    orig_shape = x.shape
    d = orig_shape[-1]
    x2 = x.reshape(-1, d)
    n_rows = x2.shape[0]
    rpb = ROWS_PER_BLOCK
    idx_arr = jnp.asarray(freq_idx, jnp.int32).reshape(1)

    row = _sc_lookup(idx_arr, freq_embeddings)

    out = pl.pallas_call(
        _tc_add_body,
        grid=(n_rows // rpb,),
        in_specs=[
            pl.BlockSpec((rpb, d), lambda i: (i, 0)),
            pl.BlockSpec((1, d), lambda i: (0, 0)),
        ],
        out_specs=pl.BlockSpec((rpb, d), lambda i: (i, 0)),
        out_shape=jax.ShapeDtypeStruct((n_rows, d), x.dtype),
        compiler_params=pltpu.CompilerParams(
            dimension_semantics=("arbitrary",),
        ),
    )(x2, row)
    return out.reshape(orig_shape)
